# Initial kernel scaffold; baseline (speedup 1.0000x reference)
import jax
import jax.numpy as jnp
from jax.experimental import pallas as pl


def kernel(x, weights):
    raise NotImplementedError("write your pallas kernel here")


# speedup vs baseline: 17.3289x; 17.3289x over previous
import functools

import jax
import jax.numpy as jnp
from jax import lax
from jax.experimental import pallas as pl
from jax.experimental.pallas import tpu as pltpu
from jax.experimental.pallas import tpu_sc as plsc

MAXP = 100000
EMB = 64
LANES = 16
SUB = 128
NSUB = 4
CHUNK = SUB * NSUB


def _make_kernel(n_total: int):
    info = plsc.get_sparse_core_info()
    nw = info.num_cores * info.num_subcores
    per_w = n_total // nw
    n_chunks = per_w // CHUNK
    assert per_w % CHUNK == 0

    mesh = plsc.VectorSubcoreMesh(core_axis_name="c", subcore_axis_name="s")

    @functools.partial(
        pl.kernel,
        mesh=mesh,
        compiler_params=pltpu.CompilerParams(use_tc_tiling_on_sc=False),
        out_type=jax.ShapeDtypeStruct((n_total, EMB), jnp.float32),
        scratch_types=[
            pltpu.VMEM((NSUB, SUB), jnp.int32),
            pltpu.VMEM((CHUNK, EMB), jnp.float32),
            pltpu.SemaphoreType.DMA,
            pltpu.SemaphoreType.DMA,
        ],
    )
    def k(x_hbm, tab_hbm, out_hbm, idx_v, rows_v, gsem, isem):
        wid = lax.axis_index("s") * info.num_cores + lax.axis_index("c")
        row0 = wid * (per_w // SUB)
        out0 = wid * per_w

        def body(g, carry):
            pltpu.async_copy(
                x_hbm.at[pl.ds(row0 + g * NSUB, NSUB)], idx_v, isem
            ).wait()
            for j in range(NSUB):
                for i in range(SUB // LANES):
                    sl = (j, pl.ds(i * LANES, LANES))
                    v = idx_v[sl]
                    v = jnp.minimum(jnp.maximum(v, -MAXP), MAXP) + MAXP
                    idx_v[sl] = v
            cps = [
                pltpu.async_copy(
                    tab_hbm.at[idx_v.at[j]],
                    rows_v.at[pl.ds(j * SUB, SUB)],
                    gsem,
                )
                for j in range(NSUB)
            ]
            for cp in cps:
                cp.wait()
            pltpu.async_copy(
                rows_v, out_hbm.at[pl.ds(out0 + g * CHUNK, CHUNK)], isem
            ).wait()
            return carry

        lax.fori_loop(0, n_chunks, body, 0)

    return k


def kernel(x, weights):
    n = x.size
    x2 = x.reshape(n // SUB, SUB)
    out = _make_kernel(n)(x2, weights)
    return out.reshape(*x.shape, EMB)


# speedup vs baseline: 18.5666x; 1.0714x over previous
import functools

import jax
import jax.numpy as jnp
from jax import lax
from jax.experimental import pallas as pl
from jax.experimental.pallas import tpu as pltpu
from jax.experimental.pallas import tpu_sc as plsc

MAXP = 100000
EMB = 64
LANES = 16
SUB = 128
NSUB = 4
CHUNK = SUB * NSUB


def _make_kernel(n_total: int):
    info = plsc.get_sparse_core_info()
    nw = info.num_cores * info.num_subcores
    per_w = n_total // nw
    n_rows = per_w // SUB
    n_chunks = per_w // CHUNK
    assert per_w % CHUNK == 0 and n_chunks % 2 == 0

    mesh = plsc.VectorSubcoreMesh(core_axis_name="c", subcore_axis_name="s")

    @functools.partial(
        pl.kernel,
        mesh=mesh,
        compiler_params=pltpu.CompilerParams(use_tc_tiling_on_sc=False),
        out_type=jax.ShapeDtypeStruct((n_total, EMB), jnp.float32),
        scratch_types=[
            pltpu.VMEM((n_rows, SUB), jnp.int32),
            pltpu.VMEM((CHUNK, EMB), jnp.float32),
            pltpu.VMEM((CHUNK, EMB), jnp.float32),
            pltpu.SemaphoreType.DMA,
            pltpu.SemaphoreType.DMA,
            pltpu.SemaphoreType.DMA,
            pltpu.SemaphoreType.DMA,
            pltpu.SemaphoreType.DMA,
        ],
    )
    def k(x_hbm, tab_hbm, out_hbm, idx_v, rows0, rows1, isem, g0, g1, o0, o1):
        wid = lax.axis_index("s") * info.num_cores + lax.axis_index("c")
        row0 = wid * n_rows
        out0 = wid * per_w
        rows = (rows0, rows1)
        gsem = (g0, g1)
        osem = (o0, o1)

        pltpu.async_copy(x_hbm.at[pl.ds(row0, n_rows)], idx_v, isem).wait()

        def pbody(r, carry):
            for i in range(SUB // LANES):
                sl = (r, pl.ds(i * LANES, LANES))
                v = idx_v[sl]
                idx_v[sl] = jnp.minimum(jnp.maximum(v, -MAXP), MAXP) + MAXP
            return carry

        lax.fori_loop(0, n_rows, pbody, 0)

        def gathers(g, b):
            for j in range(NSUB):
                pltpu.async_copy(
                    tab_hbm.at[idx_v.at[g * NSUB + j]],
                    rows[b].at[pl.ds(j * SUB, SUB)],
                    gsem[b],
                )

        def drain_gathers(b):
            pltpu.make_async_copy(
                tab_hbm.at[pl.ds(0, CHUNK)], rows[b], gsem[b]
            ).wait()

        def store(g, b):
            pltpu.async_copy(
                rows[b], out_hbm.at[pl.ds(out0 + g * CHUNK, CHUNK)], osem[b]
            )

        def drain_store(b):
            pltpu.make_async_copy(
                rows[b], out_hbm.at[pl.ds(out0, CHUNK)], osem[b]
            ).wait()

        def body(g2, carry):
            for b in range(2):
                g = g2 * 2 + b
                nb = 1 - b

                @pl.when(g >= 2)
                def _():
                    drain_store(b)

                gathers(g, b)

                @pl.when(g >= 1)
                def _():
                    drain_gathers(nb)
                    store(g - 1, nb)

            return carry

        lax.fori_loop(0, n_chunks // 2, body, 0)

        drain_gathers(1)
        store(n_chunks - 1, 1)
        drain_store(0)
        drain_store(1)

    return k


def kernel(x, weights):
    n = x.size
    x2 = x.reshape(n // SUB, SUB)
    out = _make_kernel(n)(x2, weights)
    return out.reshape(*x.shape, EMB)
